# Initial kernel scaffold; baseline (speedup 1.0000x reference)
import jax
import jax.numpy as jnp
from jax.experimental import pallas as pl


def kernel(features, samples):
    raise NotImplementedError("write your pallas kernel here")


# speedup vs baseline: 3.0670x; 3.0670x over previous
import functools

import jax
import jax.numpy as jnp
from jax import lax
from jax.experimental import pallas as pl
from jax.experimental.pallas import tpu as pltpu
from jax.experimental.pallas import tpu_sc as plsc

_N = 10000
_D = 128
_L = 16
_DV = _D // _L
_S = 51
_G = _S + 1
_PW = 2 * _G
_NC, _NS = 2, 16
_NW = _NC * _NS
_PAIRS = (_N + 1) // 2
_PPW = -(-_PAIRS // _NW)
if _PPW % 2:
    _PPW += 1
_PAIRS_PAD = _PPW * _NW
_OW = 112
_EPS = 1e-8


def _normalize_body(x_ref, o_ref):
    x = x_ref[...]
    n = jnp.sqrt(jnp.sum(x * x, axis=-1, keepdims=True))
    o_ref[...] = x / jnp.maximum(n, _EPS)


def _normalize(features):
    return pl.pallas_call(
        _normalize_body,
        grid=(10,),
        in_specs=[pl.BlockSpec((_N // 10, _D), lambda i: (i, 0))],
        out_specs=pl.BlockSpec((_N // 10, _D), lambda i: (i, 0)),
        out_shape=jax.ShapeDtypeStruct((_N, _D), jnp.float32),
    )(features)


def _sim_body(fn_hbm, idx_hbm, sims_hbm, idx_v, buf_a, buf_b, out_a, out_b,
              acc, gsem_a, gsem_b, osem_a, osem_b):
    wid = lax.axis_index("s") * _NC + lax.axis_index("c")
    pair0 = wid * _PPW

    pltpu.make_async_copy(
        idx_hbm.at[pl.ds(pair0 * _PW, _PPW * _PW)], idx_v, gsem_a).start()
    pltpu.make_async_copy(
        idx_hbm.at[pl.ds(pair0 * _PW, _PPW * _PW)], idx_v, gsem_a).wait()

    def fire_gather(p_local, buf, sem):
        idx = idx_v.at[pl.ds(p_local * _PW, _PW)]
        pltpu.make_async_copy(fn_hbm.at[idx], buf, sem).start()

    def wait_gather(p_local, buf, sem):
        idx = idx_v.at[pl.ds(p_local * _PW, _PW)]
        pltpu.make_async_copy(fn_hbm.at[idx], buf, sem).wait()

    def fire_out(out_v, p_global, sem):
        pltpu.make_async_copy(
            out_v, sims_hbm.at[pl.ds(p_global * _OW, _OW)], sem).start()

    def wait_out(out_v, sem):
        pltpu.make_async_copy(
            out_v, sims_hbm.at[pl.ds(pair0 * _OW, _OW)], sem).wait()

    def dots(buf, qbase):
        q = [buf[qbase, pl.ds(_L * k, _L)] for k in range(_DV)]

        @pl.loop(qbase, qbase + _G)
        def _(j):
            parts = [buf[j, pl.ds(_L * k, _L)] * q[k] for k in range(_DV)]
            while len(parts) > 1:
                parts = [a + b for a, b in zip(parts[::2], parts[1::2])]
            acc[j, pl.ds(0, _L)] = parts[0]

    def compute_pair(buf, out_v):
        dots(buf, 0)
        dots(buf, _G)
        lane = lax.iota(jnp.int32, _L)
        for g in range(_OW // _L):
            rows_idx = lane + (_L * g)
            tot = plsc.load_gather(acc, [rows_idx, jnp.zeros_like(lane)])
            for l in range(1, _L):
                tot = tot + plsc.load_gather(
                    acc, [rows_idx, jnp.full_like(lane, l)])
            out_v[pl.ds(_L * g, _L)] = tot

    fire_gather(0, buf_a, gsem_a)

    @pl.loop(0, _PPW // 2)
    def _(k):
        p0 = 2 * k
        p1 = p0 + 1

        @pl.when(k > 0)
        def _():
            wait_out(out_a, osem_a)
            wait_out(out_b, osem_b)

        wait_gather(p0, buf_a, gsem_a)
        fire_gather(p1, buf_b, gsem_b)
        compute_pair(buf_a, out_a)
        fire_out(out_a, pair0 + p0, osem_a)

        wait_gather(p1, buf_b, gsem_b)

        @pl.when(k < _PPW // 2 - 1)
        def _():
            fire_gather(p0 + 2, buf_a, gsem_a)

        compute_pair(buf_b, out_b)
        fire_out(out_b, pair0 + p1, osem_b)

    wait_out(out_a, osem_a)
    wait_out(out_b, osem_b)


def _similarities(fn, idx):
    k = pl.kernel(
        _sim_body,
        mesh=plsc.VectorSubcoreMesh(core_axis_name="c", subcore_axis_name="s"),
        compiler_params=pltpu.CompilerParams(needs_layout_passes=False),
        out_type=jax.ShapeDtypeStruct((_PAIRS_PAD * _OW,), jnp.float32),
        scratch_types=[
            pltpu.VMEM((_PPW * _PW,), jnp.int32),
            pltpu.VMEM((_PW, _D), jnp.float32),
            pltpu.VMEM((_PW, _D), jnp.float32),
            pltpu.VMEM((_OW,), jnp.float32),
            pltpu.VMEM((_OW,), jnp.float32),
            pltpu.VMEM((_OW, _L), jnp.float32),
            pltpu.SemaphoreType.DMA,
            pltpu.SemaphoreType.DMA,
            pltpu.SemaphoreType.DMA,
            pltpu.SemaphoreType.DMA,
        ],
    )
    return k(fn, idx)


_RB = _PAIRS_PAD // 8


def _loss_body(s_ref, o_ref):
    i = pl.program_id(0)
    x = s_ref[...]
    row = i * _RB + lax.broadcasted_iota(jnp.int32, (_RB, 1), 0)

    def half_loss(base, qrow):
        pos = x[:, base + 1:base + 2]
        neg = jnp.sum(jnp.exp(x[:, base + 2:base + _G]), axis=1, keepdims=True)
        li = jnp.log(neg + _EPS) - pos
        return jnp.where(qrow < _N, li, 0.0)

    total = jnp.sum(half_loss(0, 2 * row)) + jnp.sum(half_loss(_G, 2 * row + 1))

    @pl.when(i == 0)
    def _():
        o_ref[...] = jnp.zeros_like(o_ref)

    o_ref[...] += total

    @pl.when(i == pl.num_programs(0) - 1)
    def _():
        o_ref[...] = o_ref[...] / _N


def _loss(sims):
    out = pl.pallas_call(
        _loss_body,
        grid=(8,),
        in_specs=[pl.BlockSpec((_RB, _OW), lambda i: (i, 0))],
        out_specs=pl.BlockSpec((1, 1), lambda i: (0, 0)),
        out_shape=jax.ShapeDtypeStruct((1, 1), jnp.float32),
    )(sims)
    return out[0, 0]


def kernel(features, samples):
    features = features.astype(jnp.float32)
    samples = samples.astype(jnp.int32)

    fn = _normalize(features)

    self_col = jnp.arange(_N, dtype=jnp.int32)[:, None]
    idx = jnp.concatenate([self_col, samples], axis=1)
    idx = idx.reshape(_PAIRS, _PW)
    idx = jnp.pad(idx, ((0, _PAIRS_PAD - _PAIRS), (0, 0)))

    sims_flat = _similarities(fn, idx.reshape(-1))
    return _loss(sims_flat.reshape(_PAIRS_PAD, _OW))
